# baseline (speedup 1.0000x reference)
import functools

import jax
import jax.numpy as jnp
from jax import lax
from jax.experimental import pallas as pl
from jax.experimental.pallas import tpu as pltpu
from jax.experimental.pallas import tpu_sc as plsc

_UNKNOWN = 1
_LANES = 16


def _windows(l):
    out = []
    pos = 0
    while pos + _LANES <= l:
        out.append((pos, 0))
        pos += _LANES
    if pos < l:
        out.append((l - _LANES, _LANES - (l - pos)))
    return out


@functools.lru_cache(maxsize=None)
def _build(b, l, d, nb):
    info = plsc.get_sparse_core_info()
    nw = info.num_cores * info.num_subcores
    assert b % (nw * nb) == 0
    rows_w = b // nw
    nchunk = rows_w // nb
    wins = _windows(l)
    nq = d // _LANES

    mesh = plsc.VectorSubcoreMesh(core_axis_name="c", subcore_axis_name="s")

    @functools.partial(
        pl.kernel,
        mesh=mesh,
        compiler_params=pltpu.CompilerParams(
            needs_layout_passes=False, use_tc_tiling_on_sc=False),
        out_type=jax.ShapeDtypeStruct((l, d // 8, b // 128, 8, 128),
                                      jnp.float32),
        scratch_types=[
            pltpu.VMEM((nb, l), jnp.int32),
            pltpu.VMEM((nb, l), jnp.int32),
            pltpu.VMEM((nb, l), jnp.int32),
            pltpu.VMEM((nb, l, d), jnp.float32),
            pltpu.VMEM((l, d // 8, 8, nb), jnp.float32),
            pltpu.VMEM((d,), jnp.float32),
            pltpu.VMEM((d,), jnp.float32),
            pltpu.SemaphoreType.DMA,
        ],
    )
    def emb(gids_hbm, spec_hbm, table_hbm, unk_hbm, out_hbm,
            gids_v, spec_v, uflag_v, rows_v, trows_v, t0_v, uemb_v, sem):
        wid = lax.axis_index("s") * info.num_cores + lax.axis_index("c")
        base = wid * rows_w
        pltpu.sync_copy(unk_hbm, uemb_v)
        pltpu.sync_copy(table_hbm.at[0], t0_v)
        t0q = [t0_v[pl.ds(q * _LANES, _LANES)] for q in range(nq)]

        def chunk_body(ci, carry):
            boff = base + ci * nb
            pltpu.sync_copy(gids_hbm.at[pl.ds(boff, nb)], gids_v)
            pltpu.sync_copy(spec_hbm.at[pl.ds(boff, nb)], spec_v)

            cps = [
                pltpu.async_copy(
                    table_hbm.at[gids_v.at[i]], rows_v.at[i], sem)
                for i in range(nb)
            ]

            def detect(i, acc):
                for off, _ in wins:
                    g = gids_v[i, pl.ds(off, _LANES)]
                    unk_i = jnp.where(g == _UNKNOWN, 1, 0)
                    uflag_v[i, pl.ds(off, _LANES)] = unk_i
                    acc = acc | unk_i
                return acc

            acc = lax.fori_loop(0, nb, detect,
                                jnp.zeros((_LANES,), jnp.int32))
            any_unk = plsc.all_reduce_population_count(acc != 0)[0] > 0

            for cp in cps:
                cp.wait()

            def blend(i, c):
                for off, lane0 in wins:
                    sv = spec_v[i, pl.ds(off, _LANES)]
                    for lane in range(lane0, _LANES):
                        j = off + lane
                        m = jnp.broadcast_to(sv[lane], (_LANES,)) != 0
                        for q in range(nq):
                            sl = pl.ds(q * _LANES, _LANES)
                            rows_v[i, j, sl] = jnp.where(
                                m, t0q[q], rows_v[i, j, sl])
                return c
            lax.fori_loop(0, nb, blend, 0)

            @pl.when(any_unk)
            def _fixup():
                def fix(i, c):
                    for off, lane0 in wins:
                        g = uflag_v[i, pl.ds(off, _LANES)]

                        @pl.when(plsc.all_reduce_population_count(
                            g != 0)[0] > 0)
                        def _():
                            for lane in range(lane0, _LANES):
                                @pl.when(g[lane] == 1)
                                def _():
                                    j = off + lane
                                    for q in range(nq):
                                        sl = pl.ds(q * _LANES, _LANES)
                                        rows_v[i, j, sl] = uemb_v[sl]
                    return c
                lax.fori_loop(0, nb, fix, 0)

            biota = lax.iota(jnp.int32, _LANES)

            def xpose(j, c):
                jv = jnp.broadcast_to(j, (_LANES,))
                for fh in range(d // 8):
                    for fl in range(8):
                        fv = jnp.broadcast_to(fh * 8 + fl, (_LANES,))
                        trows_v[j, fh, fl, :] = plsc.load_gather(
                            rows_v, [biota, jv, fv])
                return c
            lax.fori_loop(0, l, xpose, 0)

            pltpu.sync_copy(
                trows_v,
                out_hbm.at[:, :, boff // 128, :, pl.ds(boff % 128, nb)])
            return carry

        lax.fori_loop(0, nchunk, chunk_body, 0)

    return emb


def kernel(grid_ids, special_pos, table, unk_emb):
    b, l = grid_ids.shape
    _, d = table.shape
    emb = _build(b, l, d, 16)
    spec = special_pos.astype(jnp.int32)
    out5 = emb(grid_ids, spec, table, unk_emb)
    return jnp.transpose(out5, (2, 4, 0, 1, 3)).reshape(b, l, d)


# speedup vs baseline: 1.6462x; 1.6462x over previous
import functools

import jax
import jax.numpy as jnp
from jax import lax
from jax.experimental import pallas as pl
from jax.experimental.pallas import tpu as pltpu
from jax.experimental.pallas import tpu_sc as plsc

_UNKNOWN = 1
_LANES = 16


def _windows(l):
    out = []
    pos = 0
    while pos + _LANES <= l:
        out.append((pos, 0))
        pos += _LANES
    if pos < l:
        out.append((l - _LANES, _LANES - (l - pos)))
    return out


@functools.lru_cache(maxsize=None)
def _build(b, l, d, nb):
    info = plsc.get_sparse_core_info()
    nw = info.num_cores * info.num_subcores
    assert b % (nw * nb) == 0
    rows_w = b // nw
    nchunk = rows_w // nb
    assert nchunk % 2 == 0
    wins = _windows(l)
    nq = d // _LANES

    mesh = plsc.VectorSubcoreMesh(core_axis_name="c", subcore_axis_name="s")

    @functools.partial(
        pl.kernel,
        mesh=mesh,
        compiler_params=pltpu.CompilerParams(
            needs_layout_passes=False, use_tc_tiling_on_sc=False),
        out_type=jax.ShapeDtypeStruct((b, l, d), jnp.float32),
        scratch_types=[
            pltpu.VMEM((2, nb, l), jnp.int32),
            pltpu.VMEM((2, nb, l), jnp.int32),
            pltpu.VMEM((2, nb, l), jnp.int32),
            pltpu.VMEM((2, nb, l, d), jnp.float32),
            pltpu.VMEM((d,), jnp.float32),
            pltpu.VMEM((d,), jnp.float32),
            pltpu.SemaphoreType.DMA,
            pltpu.SemaphoreType.DMA,
            pltpu.SemaphoreType.DMA,
            pltpu.SemaphoreType.DMA,
        ],
    )
    def emb(gids_hbm, spec_hbm, table_hbm, unk_hbm, out_hbm,
            gids_v, spec_v, uflag_v, rows_v, t0_v, uemb_v,
            gsem0, gsem1, osem0, osem1):
        wid = lax.axis_index("s") * info.num_cores + lax.axis_index("c")
        base = wid * rows_w
        gsems = (gsem0, gsem1)
        osems = (osem0, osem1)
        pltpu.sync_copy(unk_hbm, uemb_v)
        pltpu.sync_copy(table_hbm.at[0], t0_v)
        t0q = [t0_v[pl.ds(q * _LANES, _LANES)] for q in range(nq)]

        def start(slot, ci):
            boff = base + ci * nb
            pltpu.sync_copy(gids_hbm.at[pl.ds(boff, nb)], gids_v.at[slot])
            pltpu.sync_copy(spec_hbm.at[pl.ds(boff, nb)], spec_v.at[slot])
            for i in range(nb):
                pltpu.async_copy(
                    table_hbm.at[gids_v.at[slot, i]],
                    rows_v.at[slot, i], gsems[slot])

        def finish(slot, ci):
            boff = base + ci * nb

            def detect(i, acc):
                for off, _ in wins:
                    g = gids_v[slot, i, pl.ds(off, _LANES)]
                    unk_i = jnp.where(g == _UNKNOWN, 1, 0)
                    uflag_v[slot, i, pl.ds(off, _LANES)] = unk_i
                    acc = acc | unk_i
                return acc

            acc = lax.fori_loop(0, nb, detect,
                                jnp.zeros((_LANES,), jnp.int32))
            any_unk = plsc.all_reduce_population_count(acc != 0)[0] > 0

            for i in range(nb):
                pltpu.make_async_copy(
                    table_hbm.at[gids_v.at[slot, i]],
                    rows_v.at[slot, i], gsems[slot]).wait()

            def blend(i, c):
                for off, lane0 in wins:
                    sv = spec_v[slot, i, pl.ds(off, _LANES)]
                    for lane in range(lane0, _LANES):
                        j = off + lane
                        m = jnp.broadcast_to(sv[lane], (_LANES,)) != 0
                        for q in range(nq):
                            sl = pl.ds(q * _LANES, _LANES)
                            rows_v[slot, i, j, sl] = jnp.where(
                                m, t0q[q], rows_v[slot, i, j, sl])
                return c
            lax.fori_loop(0, nb, blend, 0)

            @pl.when(any_unk)
            def _fixup():
                def fix(i, c):
                    for off, lane0 in wins:
                        g = uflag_v[slot, i, pl.ds(off, _LANES)]

                        @pl.when(plsc.all_reduce_population_count(
                            g != 0)[0] > 0)
                        def _():
                            for lane in range(lane0, _LANES):
                                @pl.when(g[lane] == 1)
                                def _():
                                    j = off + lane
                                    for q in range(nq):
                                        sl = pl.ds(q * _LANES, _LANES)
                                        rows_v[slot, i, j, sl] = uemb_v[sl]
                    return c
                lax.fori_loop(0, nb, fix, 0)

            pltpu.async_copy(rows_v.at[slot],
                             out_hbm.at[pl.ds(boff, nb)], osems[slot])

        def drain_out(slot, ci):
            boff = base + ci * nb
            pltpu.make_async_copy(rows_v.at[slot],
                                  out_hbm.at[pl.ds(boff, nb)],
                                  osems[slot]).wait()

        start(0, 0)

        def body(ci2, carry):
            for par in (0, 1):
                ci = ci2 * 2 + par
                nxt = 1 - par

                @pl.when(ci + 1 < nchunk)
                def _():
                    @pl.when(ci >= 1)
                    def _():
                        drain_out(nxt, ci - 1)
                    start(nxt, ci + 1)

                finish(par, ci)
            return carry

        lax.fori_loop(0, nchunk // 2, body, 0)
        drain_out(nchunk % 2, nchunk - 2)
        drain_out(1 - nchunk % 2, nchunk - 1)

    return emb


def kernel(grid_ids, special_pos, table, unk_emb):
    b, l = grid_ids.shape
    _, d = table.shape
    emb = _build(b, l, d, 16)
    spec = special_pos.astype(jnp.int32)
    return emb(grid_ids, spec, table, unk_emb)
